# Initial kernel scaffold; baseline (speedup 1.0000x reference)
import numpy as np
import jax
import jax.numpy as jnp
from jax.experimental import pallas as pl
from jax.experimental.pallas import tpu as pltpu

_H = 16
_OLD = 2 * 24 - 1
_NEW = 2 * 32 - 1
_S = 32 * 32 + 1


def _keys_cubic(x):
    out = ((1.5 * x - 2.5) * x) * x + 1.0
    out = np.where(x >= 1.0, ((-0.5 * x + 2.5) * x - 4.0) * x + 2.0, out)
    return np.where(x >= 2.0, 0.0, out)


def _weight_mat(in_size, out_size):
    scale = out_size / in_size
    sample_f = (np.arange(out_size, dtype=np.float64) + 0.5) / scale - 0.5
    x = np.abs(sample_f[None, :] - np.arange(in_size, dtype=np.float64)[:, None])
    w = _keys_cubic(x)
    total = w.sum(axis=0, keepdims=True)
    w = np.where(np.abs(total) > 1000.0 * float(np.finfo(np.float32).eps),
                 w / np.where(total != 0, total, 1.0), 0.0)
    ok = (sample_f >= -0.5) & (sample_f <= in_size - 0.5)
    return np.where(ok[None, :], w, 0.0)


_WMAT = _weight_mat(_OLD, _NEW)
_RH = np.ascontiguousarray(_WMAT.T).astype(np.float32)
_RW = _WMAT.astype(np.float32)

_pw = np.arange(32)[:, None]
_qw = np.arange(32)[None, :]
_C = np.zeros((_NEW, 1024), np.float32)
_C[(_pw - _qw + 31).reshape(-1), np.arange(1024)] = 1.0


def _body(delta_ref, brd_ref, old_ref, out_ref):
    d = delta_ref[0, 0]
    h = pl.program_id(0)
    old = old_ref[0]
    rh = jnp.asarray(_RH)
    rw = jnp.asarray(_RW)
    img = jnp.dot(rh, jnp.dot(old, rw, preferred_element_type=jnp.float32),
                  preferred_element_type=jnp.float32) + d
    c = jnp.asarray(_C)
    for qh in range(32):
        n = jax.lax.slice(img, (31 - qh, 0), (63 - qh, _NEW))
        blk = jnp.dot(n, c, preferred_element_type=jnp.float32)
        out_ref[0, 1:, 1 + 32 * qh: 33 + 32 * qh] = blk.reshape(1024, 32)
    v1 = brd_ref[h, 0] + d
    v2 = brd_ref[h, 1] + d
    v3 = brd_ref[h, 2] + d
    col = jax.lax.broadcasted_iota(jnp.int32, (1, _S), 1)
    out_ref[0, 0:1, :] = jnp.where(col == 0, v3, v1)
    out_ref[0, 1:, 0:1] = jnp.full((_S - 1, 1), v2, jnp.float32)


def kernel(relative_position_bias_table, training_window_size):
    tab = relative_position_bias_table
    tws = training_window_size
    delta = jnp.sum(tws - jnp.asarray((32, 32), dtype=tws.dtype)).astype(tab.dtype)
    delta2 = jnp.reshape(delta, (1, 1))
    old = tab[:-3, :].T.reshape(_H, _OLD, _OLD)
    brd = tab[-3:, :].T
    out = pl.pallas_call(
        _body,
        grid=(_H,),
        in_specs=[
            pl.BlockSpec(memory_space=pltpu.SMEM),
            pl.BlockSpec(memory_space=pltpu.SMEM),
            pl.BlockSpec((1, _OLD, _OLD), lambda h: (h, 0, 0)),
        ],
        out_specs=pl.BlockSpec((1, _S, _S), lambda h: (h, 0, 0)),
        out_shape=jax.ShapeDtypeStruct((_H, _S, _S), jnp.float32),
    )(delta2, brd, old)
    return out


# speedup vs baseline: 32.8625x; 32.8625x over previous
import numpy as np
import jax
import jax.numpy as jnp
from jax.experimental import pallas as pl
from jax.experimental.pallas import tpu as pltpu

_H = 16
_OLD = 2 * 24 - 1
_NEW = 2 * 32 - 1
_S = 32 * 32 + 1


def _keys_cubic(x):
    out = ((1.5 * x - 2.5) * x) * x + 1.0
    out = np.where(x >= 1.0, ((-0.5 * x + 2.5) * x - 4.0) * x + 2.0, out)
    return np.where(x >= 2.0, 0.0, out)


def _weight_mat(in_size, out_size):
    scale = out_size / in_size
    sample_f = (np.arange(out_size, dtype=np.float64) + 0.5) / scale - 0.5
    x = np.abs(sample_f[None, :] - np.arange(in_size, dtype=np.float64)[:, None])
    w = _keys_cubic(x)
    total = w.sum(axis=0, keepdims=True)
    w = np.where(np.abs(total) > 1000.0 * float(np.finfo(np.float32).eps),
                 w / np.where(total != 0, total, 1.0), 0.0)
    ok = (sample_f >= -0.5) & (sample_f <= in_size - 0.5)
    return np.where(ok[None, :], w, 0.0)


_WMAT = _weight_mat(_OLD, _NEW)
_RH = np.ascontiguousarray(_WMAT.T).astype(np.float32)
_RWREV = np.zeros((_OLD, 128), np.float32)
_RWREV[:, :_NEW] = _WMAT[:, ::-1]


def _body(delta_ref, brd_ref, old_ref, rh_ref, rw_ref, out_ref):
    d = delta_ref[0, 0]
    h = pl.program_id(0)
    old = old_ref[0]
    rh = rh_ref[...]
    rw = rw_ref[...]
    imgrev = jnp.dot(rh, jnp.dot(old, rw, preferred_element_type=jnp.float32),
                     preferred_element_type=jnp.float32) + d
    for qh in range(32):
        nrev = jax.lax.slice(imgrev, (31 - qh, 0), (63 - qh, 128))
        nexp = jnp.broadcast_to(nrev[:, None, :], (32, 32, 128))
        rolled = pltpu.roll(nexp, 97, 2, stride=1, stride_axis=1)
        colblk = rolled[:, :, :32].reshape(1024, 32)
        out_ref[0, 1:, 1 + 32 * qh: 33 + 32 * qh] = colblk
    v1 = brd_ref[h, 0] + d
    v2 = brd_ref[h, 1] + d
    v3 = brd_ref[h, 2] + d
    col = jax.lax.broadcasted_iota(jnp.int32, (1, _S), 1)
    out_ref[0, 0:1, :] = jnp.where(col == 0, v3, v1)
    out_ref[0, 1:, 0:1] = jnp.full((_S - 1, 1), v2, jnp.float32)


def kernel(relative_position_bias_table, training_window_size):
    tab = relative_position_bias_table
    tws = training_window_size
    delta = jnp.sum(tws - jnp.asarray((32, 32), dtype=tws.dtype)).astype(tab.dtype)
    delta2 = jnp.reshape(delta, (1, 1))
    old = tab[:-3, :].T.reshape(_H, _OLD, _OLD)
    brd = tab[-3:, :].T
    out = pl.pallas_call(
        _body,
        grid=(_H,),
        in_specs=[
            pl.BlockSpec(memory_space=pltpu.SMEM),
            pl.BlockSpec(memory_space=pltpu.SMEM),
            pl.BlockSpec((1, _OLD, _OLD), lambda h: (h, 0, 0)),
            pl.BlockSpec((_NEW, _OLD), lambda h: (0, 0)),
            pl.BlockSpec((_OLD, 128), lambda h: (0, 0)),
        ],
        out_specs=pl.BlockSpec((1, _S, _S), lambda h: (h, 0, 0)),
        out_shape=jax.ShapeDtypeStruct((_H, _S, _S), jnp.float32),
    )(delta2, brd, old, jnp.asarray(_RH), jnp.asarray(_RWREV))
    return out
